# Initial kernel scaffold; baseline (speedup 1.0000x reference)
import jax
import jax.numpy as jnp
from jax.experimental import pallas as pl


def kernel(x, table):
    raise NotImplementedError("write your pallas kernel here")


# speedup vs baseline: 1.4933x; 1.4933x over previous
import functools

import jax
import jax.numpy as jnp
from jax import lax
from jax.experimental import pallas as pl
from jax.experimental.pallas import tpu as pltpu
from jax.experimental.pallas import tpu_sc as plsc

_CHUNK = 128
_NBUF = 4


@functools.lru_cache(maxsize=None)
def _build_gather(N, V, D, num_cores, num_subcores):
    n_workers = num_cores * num_subcores
    b_per_w = N // n_workers
    n_chunks = b_per_w // _CHUNK
    n_groups = n_chunks // _NBUF
    assert b_per_w * n_workers == N
    assert n_groups * _NBUF == n_chunks

    mesh = plsc.VectorSubcoreMesh(core_axis_name="c", subcore_axis_name="s")

    @functools.partial(
        pl.kernel,
        mesh=mesh,
        compiler_params=pltpu.CompilerParams(use_tc_tiling_on_sc=False),
        out_type=jax.ShapeDtypeStruct((N, D), jnp.float32),
        scratch_types=[
            pltpu.VMEM((b_per_w,), jnp.int32),
            pltpu.VMEM((_NBUF, _CHUNK, D), jnp.float32),
            pltpu.SemaphoreType.DMA,
        ],
    )
    def gather_kernel(table_hbm, idx_hbm, out_hbm, idx_v, rows_v, sem):
        wid = lax.axis_index("s") * num_cores + lax.axis_index("c")
        base = wid * b_per_w
        pltpu.sync_copy(idx_hbm.at[pl.ds(base, b_per_w)], idx_v)

        def group(g, carry):
            handles = []
            for b in range(_NBUF):
                off = (g * _NBUF + b) * _CHUNK
                handles.append(pltpu.async_copy(
                    table_hbm.at[idx_v.at[pl.ds(off, _CHUNK)]],
                    rows_v.at[b], sem))
            for b in range(_NBUF):
                off = (g * _NBUF + b) * _CHUNK
                handles[b].wait()
                pltpu.sync_copy(rows_v.at[b],
                                out_hbm.at[pl.ds(base + off, _CHUNK)])
            return carry

        lax.fori_loop(0, n_groups, group, 0)

    return gather_kernel


def kernel(x, table):
    bsz, hist = x.shape
    vocab, dim = table.shape
    n = bsz * hist
    idx = x.reshape(n).astype(jnp.int32)
    info = plsc.get_sparse_core_info()
    out = _build_gather(n, vocab, dim, info.num_cores, info.num_subcores)(
        table, idx)
    return out.reshape(bsz, hist, dim)


# speedup vs baseline: 1.5127x; 1.0130x over previous
import functools

import jax
import jax.numpy as jnp
from jax import lax
from jax.experimental import pallas as pl
from jax.experimental.pallas import tpu as pltpu
from jax.experimental.pallas import tpu_sc as plsc

_CHUNK = 128
_NBUF = 8


@functools.lru_cache(maxsize=None)
def _build_gather(N, V, D, num_cores, num_subcores):
    n_workers = num_cores * num_subcores
    b_per_w = N // n_workers
    n_chunks = b_per_w // _CHUNK
    n_groups = n_chunks // _NBUF
    assert b_per_w * n_workers == N
    assert n_groups * _NBUF == n_chunks

    mesh = plsc.VectorSubcoreMesh(core_axis_name="c", subcore_axis_name="s")

    @functools.partial(
        pl.kernel,
        mesh=mesh,
        compiler_params=pltpu.CompilerParams(use_tc_tiling_on_sc=False),
        out_type=jax.ShapeDtypeStruct((N, D), jnp.float32),
        scratch_types=[
            pltpu.VMEM((b_per_w,), jnp.int32),
            pltpu.VMEM((_NBUF, _CHUNK, D), jnp.float32),
            pltpu.SemaphoreType.DMA,
            pltpu.SemaphoreType.DMA,
        ],
    )
    def gather_kernel(table_hbm, idx_hbm, out_hbm, idx_v, rows_v, gsem, wsem):
        wid = lax.axis_index("s") * num_cores + lax.axis_index("c")
        base = wid * b_per_w
        pltpu.sync_copy(idx_hbm.at[pl.ds(base, b_per_w)], idx_v)

        def fire_gather(chunk, b):
            pltpu.async_copy(
                table_hbm.at[idx_v.at[pl.ds(chunk * _CHUNK, _CHUNK)]],
                rows_v.at[b], gsem)

        def wait_gather(b):
            pltpu.make_async_copy(
                table_hbm.at[idx_v.at[pl.ds(0, _CHUNK)]],
                rows_v.at[b], gsem).wait()

        def fire_writeback(chunk, b):
            pltpu.async_copy(rows_v.at[b],
                             out_hbm.at[pl.ds(base + chunk * _CHUNK, _CHUNK)],
                             wsem)

        def wait_writeback(b):
            pltpu.make_async_copy(rows_v.at[b],
                                  out_hbm.at[pl.ds(base, _CHUNK)],
                                  wsem).wait()

        for b in range(_NBUF):
            fire_gather(b, b)

        def group(g, carry):
            for b in range(_NBUF):
                wait_gather(b)
                fire_writeback(g * _NBUF + b, b)
            for b in range(_NBUF):
                wait_writeback(b)
                fire_gather((g + 1) * _NBUF + b, b)
            return carry

        lax.fori_loop(0, n_groups - 1, group, 0)

        g = n_groups - 1
        for b in range(_NBUF):
            wait_gather(b)
            fire_writeback(g * _NBUF + b, b)
        for b in range(_NBUF):
            wait_writeback(b)

    return gather_kernel


def kernel(x, table):
    bsz, hist = x.shape
    vocab, dim = table.shape
    n = bsz * hist
    idx = x.reshape(n).astype(jnp.int32)
    info = plsc.get_sparse_core_info()
    out = _build_gather(n, vocab, dim, info.num_cores, info.num_subcores)(
        table, idx)
    return out.reshape(bsz, hist, dim)
